# baseline (speedup 1.0000x reference)
import jax
import jax.numpy as jnp
from jax.experimental import pallas as pl
from jax.experimental.pallas import tpu as pltpu

_EPS_SQ = 1e-24


def _l2n(x):
    return x * jax.lax.rsqrt(
        jnp.maximum(jnp.sum(x * x, axis=-1, keepdims=True), _EPS_SQ))


def _transr_body(R, oh_ref, h_ref, pt_ref, nt_ref, relt_ref, w_ref,
                 r_out, h_out, pt_out, nt_out):
    oh = oh_ref[...].astype(jnp.bfloat16)
    r_out[...] = jnp.dot(oh, _l2n(relt_ref[...]).astype(jnp.bfloat16),
                         preferred_element_type=jnp.float32)

    masks = [oh[:, r:r + 1] for r in range(R)]

    def transr(e_ref):
        e = e_ref[...].astype(jnp.bfloat16)
        exp = jnp.concatenate([e * m for m in masks], axis=1)
        proj = jnp.dot(exp, w_ref[...], preferred_element_type=jnp.float32)
        return _l2n(proj)

    h_out[...] = transr(h_ref)
    pt_out[...] = transr(pt_ref)
    nt_out[...] = transr(nt_ref)


def _cf_body(ip_ref, ep_ref, in_ref, en_ref, pc_out, pcb_out, nc_out):
    pc = ip_ref[...] + ep_ref[...]
    pc_out[...] = pc
    pcb_out[...] = pc.astype(jnp.bfloat16)
    nc_out[...] = in_ref[...] + en_ref[...]


def _pred_body(u_ref, p_ref, o_ref):
    o_ref[...] = jax.lax.dot_general(
        u_ref[...].astype(jnp.bfloat16), p_ref[...],
        (((1,), (1,)), ((), ())), preferred_element_type=jnp.float32)


def _tile(n, target):
    t = target
    while t > 8 and n % t:
        t //= 2
    return t


def kernel(user_embed, item_embed, kg_entity_embed, kg_relation_embed,
           trans_W, users, pos_items, neg_items, heads, relations,
           pos_tails, neg_tails):
    B = int(users.shape[0])
    R, D, K = (int(s) for s in trans_W.shape)
    bf = jnp.bfloat16

    idx3 = jnp.concatenate([heads, pos_tails, neg_tails])
    ent3_rows = jnp.take(kg_entity_embed, idx3, axis=0, mode="clip")
    idx2 = jnp.concatenate([pos_items, neg_items])
    item_rows = jnp.take(item_embed, idx2, axis=0, mode="clip")
    entcf_rows = jnp.take(kg_entity_embed, idx2, axis=0, mode="clip")
    u_e = jnp.take(user_embed, users, axis=0, mode="clip")

    onehot = (relations.astype(jnp.int32)[:, None]
              == jnp.arange(R, dtype=jnp.int32)[None, :]).astype(jnp.float32)
    w_stack = trans_W.reshape(R * D, K).astype(bf)

    TB = _tile(B, 512)
    G = B // TB
    row_d = pl.BlockSpec((TB, D), lambda i: (i, 0))
    row_k = pl.BlockSpec((TB, K), lambda i: (i, 0))

    def seg(j):
        return pl.BlockSpec((TB, D), lambda i, j=j: (i + j * G, 0))

    r_o, h_o, pt_o, nt_o = pl.pallas_call(
        lambda *refs: _transr_body(R, *refs),
        grid=(G,),
        in_specs=[
            pl.BlockSpec((TB, R), lambda i: (i, 0)),
            seg(0), seg(1), seg(2),
            pl.BlockSpec((R, K), lambda i: (0, 0)),
            pl.BlockSpec((R * D, K), lambda i: (0, 0)),
        ],
        out_specs=(row_k, row_k, row_k, row_k),
        out_shape=(
            jax.ShapeDtypeStruct((B, K), jnp.float32),
            jax.ShapeDtypeStruct((B, K), jnp.float32),
            jax.ShapeDtypeStruct((B, K), jnp.float32),
            jax.ShapeDtypeStruct((B, K), jnp.float32),
        ),
        compiler_params=pltpu.CompilerParams(
            dimension_semantics=("parallel",)),
    )(onehot, ent3_rows, ent3_rows, ent3_rows, kg_relation_embed, w_stack)

    pos_comb, pos_comb_bf, neg_comb = pl.pallas_call(
        _cf_body,
        grid=(G,),
        in_specs=[seg(0), seg(0), seg(1), seg(1)],
        out_specs=(row_d, row_d, row_d),
        out_shape=(
            jax.ShapeDtypeStruct((B, D), jnp.float32),
            jax.ShapeDtypeStruct((B, D), bf),
            jax.ShapeDtypeStruct((B, D), jnp.float32),
        ),
        compiler_params=pltpu.CompilerParams(
            dimension_semantics=("parallel",)),
    )(item_rows, entcf_rows, item_rows, entcf_rows)

    preds = pl.pallas_call(
        _pred_body,
        grid=(G,),
        in_specs=[
            pl.BlockSpec((TB, D), lambda i: (i, 0)),
            pl.BlockSpec((B, D), lambda i: (0, 0)),
        ],
        out_specs=pl.BlockSpec((TB, B), lambda i: (i, 0)),
        out_shape=jax.ShapeDtypeStruct((B, B), jnp.float32),
        compiler_params=pltpu.CompilerParams(
            dimension_semantics=("parallel",)),
    )(u_e, pos_comb_bf)

    return (u_e, pos_comb, neg_comb, h_o, r_o, pt_o, nt_o, preds)


# speedup vs baseline: 1.0837x; 1.0837x over previous
import jax
import jax.numpy as jnp
from jax.experimental import pallas as pl
from jax.experimental.pallas import tpu as pltpu

_EPS_SQ = 1e-24


def _l2n(x):
    return x * jax.lax.rsqrt(
        jnp.maximum(jnp.sum(x * x, axis=-1, keepdims=True), _EPS_SQ))


def _cf_body(ip_ref, ep_ref, in_ref, en_ref, pc_out, pcb_out, nc_out):
    pc = ip_ref[...] + ep_ref[...]
    pc_out[...] = pc
    pcb_out[...] = pc.astype(jnp.bfloat16)
    nc_out[...] = in_ref[...] + en_ref[...]


def _fused_body(R, oh_ref, h_ref, pt_ref, nt_ref, u_ref, pcb_ref, relt_ref,
                w_ref, r_out, h_out, pt_out, nt_out, pred_out):
    pred_out[...] = jax.lax.dot_general(
        u_ref[...].astype(jnp.bfloat16), pcb_ref[...],
        (((1,), (1,)), ((), ())), preferred_element_type=jnp.float32)

    oh = oh_ref[...].astype(jnp.bfloat16)
    r_out[...] = jnp.dot(oh, _l2n(relt_ref[...]).astype(jnp.bfloat16),
                         preferred_element_type=jnp.float32)

    masks = [oh[:, r:r + 1] for r in range(R)]

    def transr(e_ref):
        e = e_ref[...].astype(jnp.bfloat16)
        exp = jnp.concatenate([e * m for m in masks], axis=1)
        proj = jnp.dot(exp, w_ref[...], preferred_element_type=jnp.float32)
        return _l2n(proj)

    h_out[...] = transr(h_ref)
    pt_out[...] = transr(pt_ref)
    nt_out[...] = transr(nt_ref)


def _tile(n, target):
    t = target
    while t > 8 and n % t:
        t //= 2
    return t


def kernel(user_embed, item_embed, kg_entity_embed, kg_relation_embed,
           trans_W, users, pos_items, neg_items, heads, relations,
           pos_tails, neg_tails):
    B = int(users.shape[0])
    R, D, K = (int(s) for s in trans_W.shape)
    bf = jnp.bfloat16

    idx2 = jnp.concatenate([pos_items, neg_items])
    item_rows = jnp.take(item_embed, idx2, axis=0, mode="clip")
    entcf_rows = jnp.take(kg_entity_embed, idx2, axis=0, mode="clip")
    u_e = jnp.take(user_embed, users, axis=0, mode="clip")
    idx3 = jnp.concatenate([heads, pos_tails, neg_tails])
    ent3_rows = jnp.take(kg_entity_embed, idx3, axis=0, mode="clip")

    onehot = (relations.astype(jnp.int32)[:, None]
              == jnp.arange(R, dtype=jnp.int32)[None, :]).astype(jnp.float32)
    w_stack = trans_W.reshape(R * D, K).astype(bf)

    TB = _tile(B, 512)
    G = B // TB
    row_d = pl.BlockSpec((TB, D), lambda i: (i, 0))
    row_k = pl.BlockSpec((TB, K), lambda i: (i, 0))

    def seg(j):
        return pl.BlockSpec((TB, D), lambda i, j=j: (i + j * G, 0))

    pos_comb, pos_comb_bf, neg_comb = pl.pallas_call(
        _cf_body,
        grid=(G,),
        in_specs=[seg(0), seg(0), seg(1), seg(1)],
        out_specs=(row_d, row_d, row_d),
        out_shape=(
            jax.ShapeDtypeStruct((B, D), jnp.float32),
            jax.ShapeDtypeStruct((B, D), bf),
            jax.ShapeDtypeStruct((B, D), jnp.float32),
        ),
        compiler_params=pltpu.CompilerParams(
            dimension_semantics=("arbitrary",)),
    )(item_rows, entcf_rows, item_rows, entcf_rows)

    r_o, h_o, pt_o, nt_o, preds = pl.pallas_call(
        lambda *refs: _fused_body(R, *refs),
        grid=(G,),
        in_specs=[
            pl.BlockSpec((TB, R), lambda i: (i, 0)),
            seg(0), seg(1), seg(2),
            pl.BlockSpec((TB, D), lambda i: (i, 0)),
            pl.BlockSpec((B, D), lambda i: (0, 0)),
            pl.BlockSpec((R, K), lambda i: (0, 0)),
            pl.BlockSpec((R * D, K), lambda i: (0, 0)),
        ],
        out_specs=(row_k, row_k, row_k, row_k,
                   pl.BlockSpec((TB, B), lambda i: (i, 0))),
        out_shape=(
            jax.ShapeDtypeStruct((B, K), jnp.float32),
            jax.ShapeDtypeStruct((B, K), jnp.float32),
            jax.ShapeDtypeStruct((B, K), jnp.float32),
            jax.ShapeDtypeStruct((B, K), jnp.float32),
            jax.ShapeDtypeStruct((B, B), jnp.float32),
        ),
        compiler_params=pltpu.CompilerParams(
            dimension_semantics=("arbitrary",)),
    )(onehot, ent3_rows, ent3_rows, ent3_rows, u_e, pos_comb_bf,
      kg_relation_embed, w_stack)

    return (u_e, pos_comb, neg_comb, h_o, r_o, pt_o, nt_o, preds)


# speedup vs baseline: 1.0848x; 1.0010x over previous
import jax
import jax.numpy as jnp
from jax.experimental import pallas as pl
from jax.experimental.pallas import tpu as pltpu

_EPS_SQ = 1e-24


def _l2n(x):
    return x * jax.lax.rsqrt(
        jnp.maximum(jnp.sum(x * x, axis=-1, keepdims=True), _EPS_SQ))


def _cf_body(ip_ref, ep_ref, in_ref, en_ref, pc_out, pcb_out, nc_out):
    pc = ip_ref[...] + ep_ref[...]
    pc_out[...] = pc
    pcb_out[...] = pc.astype(jnp.bfloat16)
    nc_out[...] = in_ref[...] + en_ref[...]


def _fused_body(R, oh_ref, h_ref, pt_ref, nt_ref, u_ref, pcb_ref, relt_ref,
                w_ref, r_out, h_out, pt_out, nt_out, pred_out):
    pred_out[...] = jax.lax.dot_general(
        u_ref[...].astype(jnp.bfloat16), pcb_ref[...],
        (((1,), (1,)), ((), ())), preferred_element_type=jnp.float32)

    oh = oh_ref[...].astype(jnp.bfloat16)
    r_out[...] = jnp.dot(oh, _l2n(relt_ref[...]).astype(jnp.bfloat16),
                         preferred_element_type=jnp.float32)

    masks = [oh[:, r:r + 1] for r in range(R)]

    def transr(e_ref):
        e = e_ref[...].astype(jnp.bfloat16)
        exp = jnp.concatenate([e * m for m in masks], axis=1)
        proj = jnp.dot(exp, w_ref[...], preferred_element_type=jnp.float32)
        return _l2n(proj)

    h_out[...] = transr(h_ref)
    pt_out[...] = transr(pt_ref)
    nt_out[...] = transr(nt_ref)


def _tile(n, target):
    t = target
    while t > 8 and n % t:
        t //= 2
    return t


def kernel(user_embed, item_embed, kg_entity_embed, kg_relation_embed,
           trans_W, users, pos_items, neg_items, heads, relations,
           pos_tails, neg_tails):
    B = int(users.shape[0])
    R, D, K = (int(s) for s in trans_W.shape)
    bf = jnp.bfloat16

    idx2 = jnp.concatenate([pos_items, neg_items])
    item_rows = jnp.take(item_embed, idx2, axis=0, mode="clip")
    entcf_rows = jnp.take(kg_entity_embed, idx2, axis=0, mode="clip")
    idx3 = jnp.concatenate([heads, pos_tails, neg_tails])
    ent3_rows = jnp.take(kg_entity_embed, idx3, axis=0, mode="clip")
    u_e = jnp.take(user_embed, users, axis=0, mode="clip")

    onehot = (relations.astype(jnp.int32)[:, None]
              == jnp.arange(R, dtype=jnp.int32)[None, :]).astype(jnp.float32)
    w_stack = trans_W.reshape(R * D, K).astype(bf)

    TB = _tile(B, 512)
    G = B // TB
    row_d = pl.BlockSpec((TB, D), lambda i: (i, 0))
    row_k = pl.BlockSpec((TB, K), lambda i: (i, 0))

    def seg(j):
        return pl.BlockSpec((TB, D), lambda i, j=j: (i + j * G, 0))

    pos_comb, pos_comb_bf, neg_comb = pl.pallas_call(
        _cf_body,
        grid=(G,),
        in_specs=[seg(0), seg(0), seg(1), seg(1)],
        out_specs=(row_d, row_d, row_d),
        out_shape=(
            jax.ShapeDtypeStruct((B, D), jnp.float32),
            jax.ShapeDtypeStruct((B, D), bf),
            jax.ShapeDtypeStruct((B, D), jnp.float32),
        ),
        compiler_params=pltpu.CompilerParams(
            dimension_semantics=("arbitrary",)),
    )(item_rows, entcf_rows, item_rows, entcf_rows)

    r_o, h_o, pt_o, nt_o, preds = pl.pallas_call(
        lambda *refs: _fused_body(R, *refs),
        grid=(G,),
        in_specs=[
            pl.BlockSpec((TB, R), lambda i: (i, 0)),
            seg(0), seg(1), seg(2),
            pl.BlockSpec((TB, D), lambda i: (i, 0)),
            pl.BlockSpec((B, D), lambda i: (0, 0)),
            pl.BlockSpec((R, K), lambda i: (0, 0)),
            pl.BlockSpec((R * D, K), lambda i: (0, 0)),
        ],
        out_specs=(row_k, row_k, row_k, row_k,
                   pl.BlockSpec((TB, B), lambda i: (i, 0))),
        out_shape=(
            jax.ShapeDtypeStruct((B, K), jnp.float32),
            jax.ShapeDtypeStruct((B, K), jnp.float32),
            jax.ShapeDtypeStruct((B, K), jnp.float32),
            jax.ShapeDtypeStruct((B, K), jnp.float32),
            jax.ShapeDtypeStruct((B, B), jnp.float32),
        ),
        compiler_params=pltpu.CompilerParams(
            dimension_semantics=("arbitrary",)),
    )(onehot, ent3_rows, ent3_rows, ent3_rows, u_e, pos_comb_bf,
      kg_relation_embed, w_stack)

    return (u_e, pos_comb, neg_comb, h_o, r_o, pt_o, nt_o, preds)


# speedup vs baseline: 1.0855x; 1.0007x over previous
import jax
import jax.numpy as jnp
from jax.experimental import pallas as pl
from jax.experimental.pallas import tpu as pltpu

_EPS_SQ = 1e-24


def _l2n(x):
    return x * jax.lax.rsqrt(
        jnp.maximum(jnp.sum(x * x, axis=-1, keepdims=True), _EPS_SQ))


def _cf_body(ip_ref, ep_ref, in_ref, en_ref, pc_out, pcb_out, nc_out):
    pc = ip_ref[...] + ep_ref[...]
    pc_out[...] = pc
    pcb_out[...] = pc.astype(jnp.bfloat16)
    nc_out[...] = in_ref[...] + en_ref[...]


def _fused_body(R, oh_ref, h_ref, pt_ref, nt_ref, u_ref, pcb_ref, relt_ref,
                w_ref, r_out, h_out, pt_out, nt_out, pred_out):
    pred_out[...] = jax.lax.dot_general(
        u_ref[...].astype(jnp.bfloat16), pcb_ref[...],
        (((1,), (1,)), ((), ())), preferred_element_type=jnp.float32)

    oh = oh_ref[...]
    r_out[...] = jnp.dot(oh.astype(jnp.bfloat16),
                         _l2n(relt_ref[...]).astype(jnp.bfloat16),
                         preferred_element_type=jnp.float32)

    masks = [oh[:, r:r + 1] for r in range(R)]

    def transr(e_ref):
        e = e_ref[...]
        exp = jnp.concatenate([e * m for m in masks],
                              axis=1).astype(jnp.bfloat16)
        proj = jnp.dot(exp, w_ref[...], preferred_element_type=jnp.float32)
        return _l2n(proj)

    h_out[...] = transr(h_ref)
    pt_out[...] = transr(pt_ref)
    nt_out[...] = transr(nt_ref)


def _tile(n, target):
    t = target
    while t > 8 and n % t:
        t //= 2
    return t


def kernel(user_embed, item_embed, kg_entity_embed, kg_relation_embed,
           trans_W, users, pos_items, neg_items, heads, relations,
           pos_tails, neg_tails):
    B = int(users.shape[0])
    R, D, K = (int(s) for s in trans_W.shape)
    bf = jnp.bfloat16

    idx2 = jnp.concatenate([pos_items, neg_items])
    item_rows = jnp.take(item_embed, idx2, axis=0, mode="clip")
    entcf_rows = jnp.take(kg_entity_embed, idx2, axis=0, mode="clip")
    idx3 = jnp.concatenate([heads, pos_tails, neg_tails])
    ent3_rows = jnp.take(kg_entity_embed, idx3, axis=0, mode="clip")
    u_e = jnp.take(user_embed, users, axis=0, mode="clip")

    onehot = (relations.astype(jnp.int32)[:, None]
              == jnp.arange(R, dtype=jnp.int32)[None, :]).astype(jnp.float32)
    w_stack = trans_W.reshape(R * D, K).astype(bf)

    TB = _tile(B, 512)
    G = B // TB
    row_d = pl.BlockSpec((TB, D), lambda i: (i, 0))
    row_k = pl.BlockSpec((TB, K), lambda i: (i, 0))

    def seg(j):
        return pl.BlockSpec((TB, D), lambda i, j=j: (i + j * G, 0))

    pos_comb, pos_comb_bf, neg_comb = pl.pallas_call(
        _cf_body,
        grid=(G,),
        in_specs=[seg(0), seg(0), seg(1), seg(1)],
        out_specs=(row_d, row_d, row_d),
        out_shape=(
            jax.ShapeDtypeStruct((B, D), jnp.float32),
            jax.ShapeDtypeStruct((B, D), bf),
            jax.ShapeDtypeStruct((B, D), jnp.float32),
        ),
        compiler_params=pltpu.CompilerParams(
            dimension_semantics=("arbitrary",)),
    )(item_rows, entcf_rows, item_rows, entcf_rows)

    r_o, h_o, pt_o, nt_o, preds = pl.pallas_call(
        lambda *refs: _fused_body(R, *refs),
        grid=(G,),
        in_specs=[
            pl.BlockSpec((TB, R), lambda i: (i, 0)),
            seg(0), seg(1), seg(2),
            pl.BlockSpec((TB, D), lambda i: (i, 0)),
            pl.BlockSpec((B, D), lambda i: (0, 0)),
            pl.BlockSpec((R, K), lambda i: (0, 0)),
            pl.BlockSpec((R * D, K), lambda i: (0, 0)),
        ],
        out_specs=(row_k, row_k, row_k, row_k,
                   pl.BlockSpec((TB, B), lambda i: (i, 0))),
        out_shape=(
            jax.ShapeDtypeStruct((B, K), jnp.float32),
            jax.ShapeDtypeStruct((B, K), jnp.float32),
            jax.ShapeDtypeStruct((B, K), jnp.float32),
            jax.ShapeDtypeStruct((B, K), jnp.float32),
            jax.ShapeDtypeStruct((B, B), jnp.float32),
        ),
        compiler_params=pltpu.CompilerParams(
            dimension_semantics=("arbitrary",)),
    )(onehot, ent3_rows, ent3_rows, ent3_rows, u_e, pos_comb_bf,
      kg_relation_embed, w_stack)

    return (u_e, pos_comb, neg_comb, h_o, r_o, pt_o, nt_o, preds)
